# baseline (speedup 1.0000x reference)
import functools

import jax
import jax.numpy as jnp
from jax import lax
from jax.experimental import pallas as pl
from jax.experimental.pallas import tpu as pltpu
from jax.experimental.pallas import tpu_sc as plsc

N = 10000
E = 320000
D = 128
NT = 16
NC = 2
CH = 80
EPT = E // NT
NCHUNK = EPT // CH
NPAD = 10240
RPT = NPAD // NT
HALF = NPAD // 2
DUMP = HALF
ACC_ROWS = 5248
ACC_RPT = ACC_ROWS // NT
WPT = HALF // NT
CH7 = 128
NCHUNK7 = 158
EPT7 = NCHUNK7 * CH7
EPAD7 = NT * EPT7 - E

_MESH2 = plsc.VectorSubcoreMesh(core_axis_name="c", subcore_axis_name="s")


def _deg_body(dsts_ref, zeros_ref, deg_ref, dstbuf, ones_v, deg_sh):
    c = lax.axis_index("c")
    s = lax.axis_index("s")
    base = pl.multiple_of(s * RPT, 8)
    pltpu.sync_copy(zeros_ref, deg_sh.at[pl.ds(base, RPT)])
    for k in range(CH // 16):
        ones_v[pl.ds(k * 16, 16)] = jnp.full((16,), 1.0, jnp.float32)
    pltpu.sync_copy(dsts_ref.at[c, s], dstbuf)
    plsc.subcore_barrier()

    def chunk(j, carry):
        pltpu.sync_copy(ones_v, deg_sh.at[dstbuf.at[j]], add=True)
        return carry

    lax.fori_loop(0, NCHUNK, chunk, 0)
    plsc.subcore_barrier()
    pltpu.sync_copy(deg_sh.at[pl.ds(base, RPT)], deg_ref.at[c, pl.ds(base, RPT)])


_deg_kernel = functools.partial(
    pl.kernel,
    out_type=jax.ShapeDtypeStruct((NC, NPAD), jnp.float32),
    mesh=_MESH2,
    scratch_types=[
        pltpu.VMEM((NCHUNK, CH), jnp.int32),
        pltpu.VMEM((CH,), jnp.float32),
        pltpu.VMEM_SHARED((NPAD,), jnp.float32),
    ],
)(_deg_body)


def _hs_body(x_ref, w_ref, deg_ref, hs_ref):
    h = jnp.dot(x_ref[...], w_ref[...], preferred_element_type=jnp.float32)
    hs_ref[...] = h * lax.rsqrt(deg_ref[...] + 1.0)


def _hs_call(x_flat, W, deg_flat):
    return pl.pallas_call(
        _hs_body,
        grid=(2 * N // 200,),
        in_specs=[
            pl.BlockSpec((200, D), lambda i: (i, 0)),
            pl.BlockSpec((D, D), lambda i: (0, 0)),
            pl.BlockSpec((200, 1), lambda i: (i, 0)),
        ],
        out_specs=pl.BlockSpec((200, D), lambda i: (i, 0)),
        out_shape=jax.ShapeDtypeStruct((2 * N, D), jnp.float32),
    )(x_flat, W, deg_flat)


def _acc_body(hs_ref, srcs_ref, dst0_ref, dst1_ref, zeros_ref, acc_ref,
              srcbuf, dstb, rows0, rows1, acc_sh, semg0, semg1):
    c = lax.axis_index("c")
    s = lax.axis_index("s")
    base = pl.multiple_of(s * ACC_RPT, 8)
    pltpu.sync_copy(srcs_ref.at[c, s], srcbuf)

    for p, dstp_ref in ((0, dst0_ref), (1, dst1_ref)):
        pltpu.sync_copy(dstp_ref.at[c, s], dstb)
        pltpu.sync_copy(zeros_ref, acc_sh.at[pl.ds(base, ACC_RPT)])
        plsc.subcore_barrier()

        pltpu.async_copy(hs_ref.at[srcbuf.at[0]], rows0, semg0)
        pltpu.async_copy(hs_ref.at[srcbuf.at[1]], rows1, semg1)

        def chunk(i, carry):
            for (par, rows, semg) in ((0, rows0, semg0), (1, rows1, semg1)):
                j = 2 * i + par
                pltpu.make_async_copy(hs_ref.at[srcbuf.at[j]], rows,
                                      semg).wait()
                pltpu.sync_copy(rows, acc_sh.at[dstb.at[j]], add=True)

                @pl.when(j + 2 < NCHUNK7)
                def _():
                    pltpu.async_copy(hs_ref.at[srcbuf.at[j + 2]], rows, semg)

            return carry

        lax.fori_loop(0, NCHUNK7 // 2, chunk, 0)
        plsc.subcore_barrier()
        wbase = pl.multiple_of(s * WPT, 8)
        pltpu.sync_copy(acc_sh.at[pl.ds(wbase, WPT)],
                        acc_ref.at[c, p, pl.ds(wbase, WPT)])


_acc_kernel = functools.partial(
    pl.kernel,
    out_type=jax.ShapeDtypeStruct((NC, 2, HALF, D), jnp.float32),
    mesh=_MESH2,
    scratch_types=[
        pltpu.VMEM((NCHUNK7, CH7), jnp.int32),
        pltpu.VMEM((NCHUNK7, CH7), jnp.int32),
        pltpu.VMEM((CH7, D), jnp.float32),
        pltpu.VMEM((CH7, D), jnp.float32),
        pltpu.VMEM_SHARED((ACC_ROWS, D), jnp.float32),
        pltpu.SemaphoreType.DMA,
        pltpu.SemaphoreType.DMA,
    ],
)(_acc_body)


def _fin_body(acc_ref, hs1_ref, hs2_ref, deg1_ref, deg2_ref, b_ref,
              out1_ref, out2_ref):
    bvec = b_ref[...]
    for acc, hsr, degr, outr in ((acc_ref[0], hs1_ref, deg1_ref, out1_ref),
                                 (acc_ref[1], hs2_ref, deg2_ref, out2_ref)):
        dinv = lax.rsqrt(degr[...] + 1.0)
        v = dinv * (acc + hsr[...]) + bvec
        n = jnp.sqrt(jnp.sum(v * v, axis=1, keepdims=True))
        outr[...] = v / jnp.maximum(n, 1e-12)


def _fin_call(accv, hs, deg_flat, b2):
    nb = N // 80

    return pl.pallas_call(
        _fin_body,
        grid=(nb,),
        in_specs=[
            pl.BlockSpec((2, 80, D), lambda r: (0, r, 0)),
            pl.BlockSpec((80, D), lambda r: (r, 0)),
            pl.BlockSpec((80, D), lambda r: (nb + r, 0)),
            pl.BlockSpec((80, 1), lambda r: (r, 0)),
            pl.BlockSpec((80, 1), lambda r: (nb + r, 0)),
            pl.BlockSpec((1, D), lambda r: (0, 0)),
        ],
        out_specs=[
            pl.BlockSpec((80, D), lambda r: (r, 0)),
            pl.BlockSpec((80, D), lambda r: (r, 0)),
        ],
        out_shape=[
            jax.ShapeDtypeStruct((N, D), jnp.float32),
            jax.ShapeDtypeStruct((N, D), jnp.float32),
        ],
    )(accv, hs, hs, deg_flat, deg_flat, b2)


def kernel(x1, x2, edge_index1, edge_index2, W, b):
    dst_all = jnp.stack([edge_index1[1], edge_index2[1]])
    dsts = dst_all.reshape(NC, NT, NCHUNK, CH)
    dpad = jnp.full((NC, EPAD7), DUMP, jnp.int32)
    spad = jnp.zeros((NC, EPAD7), jnp.int32)
    srcs = jnp.concatenate(
        [jnp.stack([edge_index1[0], edge_index2[0] + N]), spad],
        axis=1).reshape(NC, NT, NCHUNK7, CH7)
    dst0 = jnp.concatenate(
        [jnp.where(dst_all < HALF, dst_all, DUMP), dpad],
        axis=1).reshape(NC, NT, NCHUNK7, CH7)
    dst1 = jnp.concatenate(
        [jnp.where(dst_all >= HALF, dst_all - HALF, DUMP), dpad],
        axis=1).reshape(NC, NT, NCHUNK7, CH7)
    zeros_deg = jnp.zeros((RPT,), jnp.float32)
    zeros_rows = jnp.zeros((ACC_RPT, D), jnp.float32)

    deg = _deg_kernel(dsts, zeros_deg)
    deg_flat = deg[:, :N].reshape(2 * N, 1)
    x_flat = jnp.concatenate([x1, x2], axis=0)
    hs = _hs_call(x_flat, W, deg_flat)
    accp = _acc_kernel(hs, srcs, dst0, dst1, zeros_rows)
    accv = accp.reshape(NC, 2 * HALF, D)
    return _fin_call(accv, hs, deg_flat, b.reshape(1, D))


# speedup vs baseline: 1.6606x; 1.6606x over previous
import functools

import jax
import jax.numpy as jnp
from jax import lax
from jax.experimental import pallas as pl
from jax.experimental.pallas import tpu as pltpu
from jax.experimental.pallas import tpu_sc as plsc

N = 10000
E = 320000
D = 128
NT = 16
NC = 2
CH = 80
EPT = E // NT
NCHUNK = EPT // CH
NPAD = 10240
RPT = NPAD // NT
HALF = NPAD // 2
DUMP = HALF
ACC_ROWS = 5248
ACC_RPT = ACC_ROWS // NT
WPT = HALF // NT

_MESH2 = plsc.VectorSubcoreMesh(core_axis_name="c", subcore_axis_name="s")


def _deg_body(dsts_ref, zeros_ref, deg_ref, dstbuf, ones_v, deg_sh):
    c = lax.axis_index("c")
    s = lax.axis_index("s")
    base = pl.multiple_of(s * RPT, 8)
    pltpu.sync_copy(zeros_ref, deg_sh.at[pl.ds(base, RPT)])
    for k in range(CH // 16):
        ones_v[pl.ds(k * 16, 16)] = jnp.full((16,), 1.0, jnp.float32)
    pltpu.sync_copy(dsts_ref.at[c, s], dstbuf)
    plsc.subcore_barrier()

    def chunk(j, carry):
        pltpu.sync_copy(ones_v, deg_sh.at[dstbuf.at[j]], add=True)
        return carry

    lax.fori_loop(0, NCHUNK, chunk, 0)
    plsc.subcore_barrier()
    pltpu.sync_copy(deg_sh.at[pl.ds(base, RPT)], deg_ref.at[c, pl.ds(base, RPT)])


_deg_kernel = functools.partial(
    pl.kernel,
    out_type=jax.ShapeDtypeStruct((NC, NPAD), jnp.float32),
    mesh=_MESH2,
    scratch_types=[
        pltpu.VMEM((NCHUNK, CH), jnp.int32),
        pltpu.VMEM((CH,), jnp.float32),
        pltpu.VMEM_SHARED((NPAD,), jnp.float32),
    ],
)(_deg_body)


def _hs_body(x_ref, w_ref, deg_ref, hs_ref):
    h = jnp.dot(x_ref[...], w_ref[...], preferred_element_type=jnp.float32)
    hs_ref[...] = h * lax.rsqrt(deg_ref[...] + 1.0)


def _hs_call(x_flat, W, deg_flat):
    return pl.pallas_call(
        _hs_body,
        grid=(2 * N // 200,),
        in_specs=[
            pl.BlockSpec((200, D), lambda i: (i, 0)),
            pl.BlockSpec((D, D), lambda i: (0, 0)),
            pl.BlockSpec((200, 1), lambda i: (i, 0)),
        ],
        out_specs=pl.BlockSpec((200, D), lambda i: (i, 0)),
        out_shape=jax.ShapeDtypeStruct((2 * N, D), jnp.float32),
    )(x_flat, W, deg_flat)


def _acc_body(hs_ref, srcs_ref, dst0_ref, dst1_ref, zeros_ref, acc_ref,
              srcbuf, dstb, rows0, rows1, acc_sh, semg0, semg1):
    c = lax.axis_index("c")
    s = lax.axis_index("s")
    base = pl.multiple_of(s * ACC_RPT, 8)
    pltpu.sync_copy(srcs_ref.at[c, s], srcbuf)

    for p, dstp_ref in ((0, dst0_ref), (1, dst1_ref)):
        pltpu.sync_copy(dstp_ref.at[c, s], dstb)
        pltpu.sync_copy(zeros_ref, acc_sh.at[pl.ds(base, ACC_RPT)])
        plsc.subcore_barrier()

        pltpu.async_copy(hs_ref.at[srcbuf.at[0]], rows0, semg0)
        pltpu.async_copy(hs_ref.at[srcbuf.at[1]], rows1, semg1)

        def chunk(i, carry):
            for (par, rows, semg) in ((0, rows0, semg0), (1, rows1, semg1)):
                j = 2 * i + par
                pltpu.make_async_copy(hs_ref.at[srcbuf.at[j]], rows,
                                      semg).wait()
                pltpu.sync_copy(rows, acc_sh.at[dstb.at[j]], add=True)

                @pl.when(j + 2 < NCHUNK)
                def _():
                    pltpu.async_copy(hs_ref.at[srcbuf.at[j + 2]], rows, semg)

            return carry

        lax.fori_loop(0, NCHUNK // 2, chunk, 0)
        plsc.subcore_barrier()
        wbase = pl.multiple_of(s * WPT, 8)
        pltpu.sync_copy(acc_sh.at[pl.ds(wbase, WPT)],
                        acc_ref.at[c, p, pl.ds(wbase, WPT)])


_acc_kernel = functools.partial(
    pl.kernel,
    out_type=jax.ShapeDtypeStruct((NC, 2, HALF, D), jnp.float32),
    mesh=_MESH2,
    scratch_types=[
        pltpu.VMEM((NCHUNK, CH), jnp.int32),
        pltpu.VMEM((NCHUNK, CH), jnp.int32),
        pltpu.VMEM((CH, D), jnp.float32),
        pltpu.VMEM((CH, D), jnp.float32),
        pltpu.VMEM_SHARED((ACC_ROWS, D), jnp.float32),
        pltpu.SemaphoreType.DMA,
        pltpu.SemaphoreType.DMA,
    ],
)(_acc_body)


def _fin_body(acc_ref, hs1_ref, hs2_ref, deg1_ref, deg2_ref, b_ref,
              out1_ref, out2_ref):
    bvec = b_ref[...]
    for acc, hsr, degr, outr in ((acc_ref[0], hs1_ref, deg1_ref, out1_ref),
                                 (acc_ref[1], hs2_ref, deg2_ref, out2_ref)):
        dinv = lax.rsqrt(degr[...] + 1.0)
        v = dinv * (acc + hsr[...]) + bvec
        n = jnp.sqrt(jnp.sum(v * v, axis=1, keepdims=True))
        outr[...] = v / jnp.maximum(n, 1e-12)


def _fin_call(accv, hs, deg_flat, b2):
    nb = N // 80

    return pl.pallas_call(
        _fin_body,
        grid=(nb,),
        in_specs=[
            pl.BlockSpec((2, 80, D), lambda r: (0, r, 0)),
            pl.BlockSpec((80, D), lambda r: (r, 0)),
            pl.BlockSpec((80, D), lambda r: (nb + r, 0)),
            pl.BlockSpec((80, 1), lambda r: (r, 0)),
            pl.BlockSpec((80, 1), lambda r: (nb + r, 0)),
            pl.BlockSpec((1, D), lambda r: (0, 0)),
        ],
        out_specs=[
            pl.BlockSpec((80, D), lambda r: (r, 0)),
            pl.BlockSpec((80, D), lambda r: (r, 0)),
        ],
        out_shape=[
            jax.ShapeDtypeStruct((N, D), jnp.float32),
            jax.ShapeDtypeStruct((N, D), jnp.float32),
        ],
    )(accv, hs, hs, deg_flat, deg_flat, b2)


def kernel(x1, x2, edge_index1, edge_index2, W, b):
    dst_all = jnp.stack([edge_index1[1], edge_index2[1]])
    dsts = dst_all.reshape(NC, NT, NCHUNK, CH)
    srcs = jnp.stack([edge_index1[0], edge_index2[0] + N]).reshape(NC, NT, NCHUNK, CH)
    dst0 = jnp.where(dst_all < HALF, dst_all, DUMP).reshape(NC, NT, NCHUNK, CH)
    dst1 = jnp.where(dst_all >= HALF, dst_all - HALF, DUMP).reshape(NC, NT, NCHUNK, CH)
    zeros_deg = jnp.zeros((RPT,), jnp.float32)
    zeros_rows = jnp.zeros((ACC_RPT, D), jnp.float32)

    deg = _deg_kernel(dsts, zeros_deg)
    deg_flat = deg[:, :N].reshape(2 * N, 1)
    x_flat = jnp.concatenate([x1, x2], axis=0)
    hs = _hs_call(x_flat, W, deg_flat)
    accp = _acc_kernel(hs, srcs, dst0, dst1, zeros_rows)
    accv = accp.reshape(NC, 2 * HALF, D)
    return _fin_call(accv, hs, deg_flat, b.reshape(1, D))
